# baseline (speedup 1.0000x reference)
import functools

import jax
import jax.numpy as jnp
from jax import lax
from jax.experimental import pallas as pl
from jax.experimental.pallas import tpu as pltpu
from jax.experimental.pallas import tpu_sc as plsc

N = 10000
E = 320000
D = 128
H1 = 16
H2 = 2
NW = 32
LANES = 16

_MESH = plsc.VectorSubcoreMesh(core_axis_name="c", subcore_axis_name="s")


def _wid():
    return lax.axis_index("s") * 2 + lax.axis_index("c")


def _zero_vmem(ref, n):
    @plsc.parallel_loop(0, n // LANES, unroll=8)
    def _(i):
        ref[pl.ds(i * LANES, LANES)] = jnp.zeros((LANES,), jnp.float32)


_EPT = E // NW


@functools.partial(
    pl.kernel,
    out_type=jax.ShapeDtypeStruct((NW, N), jnp.float32),
    mesh=_MESH,
    compiler_params=pltpu.CompilerParams(needs_layout_passes=False),
    scratch_types=[
        pltpu.VMEM((_EPT,), jnp.int32),
        pltpu.VMEM((_EPT,), jnp.float32),
        pltpu.VMEM((N,), jnp.float32),
    ],
)
def _deg_kernel(dst_hbm, ew_hbm, out_hbm, dst_v, ew_v, acc_v):
    w = _wid()
    pltpu.sync_copy(dst_hbm.at[pl.ds(w * _EPT, _EPT)], dst_v)
    pltpu.sync_copy(ew_hbm.at[pl.ds(w * _EPT, _EPT)], ew_v)
    _zero_vmem(acc_v, N)

    @plsc.parallel_loop(0, _EPT // LANES, unroll=8)
    def _(i):
        d = dst_v[pl.ds(i * LANES, LANES)]
        v = ew_v[pl.ds(i * LANES, LANES)]
        plsc.addupdate_scatter(acc_v, [d], v)

    pltpu.sync_copy(acc_v, out_hbm.at[w])


_FG1 = 4
_NC1 = H1 // _FG1 * 2
_EC1 = E // _NC1
_CH1 = 4000
_NSUB1 = _EC1 // _CH1
assert _EC1 % _CH1 == 0 and _CH1 % LANES == 0


def _edge_loop_fg(nvec, fg, src_v, dst_v, ew_v, h_v, acc_v, unroll=4):
    @plsc.parallel_loop(0, nvec, unroll=unroll)
    def _(i):
        s = src_v[pl.ds(i * LANES, LANES)]
        d = dst_v[pl.ds(i * LANES, LANES)]
        v = ew_v[pl.ds(i * LANES, LANES)]
        for f in range(fg):
            g = plsc.load_gather(h_v, [s + f * N])
            plsc.addupdate_scatter(acc_v, [d + f * N], g * v)


@functools.partial(
    pl.kernel,
    out_type=jax.ShapeDtypeStruct((_NC1 * H1 * N,), jnp.float32),
    mesh=_MESH,
    compiler_params=pltpu.CompilerParams(needs_layout_passes=False),
    scratch_types=[
        pltpu.VMEM((_FG1 * N,), jnp.float32),
        pltpu.VMEM((_FG1 * N,), jnp.float32),
        pltpu.VMEM((2 * _CH1,), jnp.int32),
        pltpu.VMEM((2 * _CH1,), jnp.int32),
        pltpu.VMEM((2 * _CH1,), jnp.float32),
        pltpu.SemaphoreType.DMA,
        pltpu.SemaphoreType.DMA,
    ],
)
def _agg1_kernel(hsT_hbm, src_hbm, dst_hbm, ew_hbm, out_hbm,
                 h_v, acc_v, src_v, dst_v, ew_v, sem0, sem1):
    w = _wid()
    g = w % _FG1
    chunk = w // _FG1
    base = chunk * _EC1
    sems = (sem0, sem1)

    def fire(c):
        b = c % 2
        off = base + c * _CH1
        sl = pl.ds(b * _CH1, _CH1)
        return [
            pltpu.async_copy(src_hbm.at[pl.ds(off, _CH1)], src_v.at[sl], sems[b]),
            pltpu.async_copy(dst_hbm.at[pl.ds(off, _CH1)], dst_v.at[sl], sems[b]),
            pltpu.async_copy(ew_hbm.at[pl.ds(off, _CH1)], ew_v.at[sl], sems[b]),
        ]

    pending = {0: fire(0)}
    pltpu.sync_copy(hsT_hbm.at[pl.ds(g * _FG1 * N, _FG1 * N)], h_v)
    _zero_vmem(acc_v, _FG1 * N)
    for c in range(_NSUB1):
        if c + 1 < _NSUB1:
            pending[(c + 1) % 2] = fire(c + 1)
        for desc in pending[c % 2]:
            desc.wait()
        sl = pl.ds((c % 2) * _CH1, _CH1)
        _edge_loop_fg(_CH1 // LANES, _FG1,
                      src_v.at[sl], dst_v.at[sl], ew_v.at[sl], h_v, acc_v,
                      unroll=8)
    pltpu.sync_copy(
        acc_v, out_hbm.at[pl.ds(chunk * H1 * N + g * _FG1 * N, _FG1 * N)])


_ECH = E // NW


@functools.partial(
    pl.kernel,
    out_type=jax.ShapeDtypeStruct((NW * H2 * N,), jnp.float32),
    mesh=_MESH,
    compiler_params=pltpu.CompilerParams(needs_layout_passes=False),
    scratch_types=[
        pltpu.VMEM((H2 * N,), jnp.float32),
        pltpu.VMEM((H2 * N,), jnp.float32),
        pltpu.VMEM((_ECH,), jnp.int32),
        pltpu.VMEM((_ECH,), jnp.int32),
        pltpu.VMEM((_ECH,), jnp.float32),
        pltpu.SemaphoreType.DMA,
    ],
)
def _agg2_kernel(hs2T_hbm, src_hbm, dst_hbm, ew_hbm, out_hbm,
                 h_v, acc_v, src_v, dst_v, ew_v, sem):
    w = _wid()
    off = w * _ECH
    pending = [
        pltpu.async_copy(src_hbm.at[pl.ds(off, _ECH)], src_v, sem),
        pltpu.async_copy(dst_hbm.at[pl.ds(off, _ECH)], dst_v, sem),
        pltpu.async_copy(ew_hbm.at[pl.ds(off, _ECH)], ew_v, sem),
    ]
    pltpu.sync_copy(hs2T_hbm, h_v)
    _zero_vmem(acc_v, H2 * N)
    for desc in pending:
        desc.wait()
    _edge_loop_fg(_ECH // LANES, H2, src_v, dst_v, ew_v, h_v, acc_v, unroll=8)
    pltpu.sync_copy(acc_v, out_hbm.at[pl.ds(w * H2 * N, H2 * N)])


def _prep_body(degp_ref, x_ref, w1t_ref, dinv_ref, hsT_ref):
    deg = jnp.sum(degp_ref[...], axis=0, keepdims=True) + 1.0
    dinv = jnp.where(deg > 0, lax.rsqrt(jnp.maximum(deg, 1e-12)), 0.0)
    dinv_ref[...] = dinv
    h = lax.dot_general(w1t_ref[...], x_ref[...], (((1,), (1,)), ((), ())),
                        preferred_element_type=jnp.float32,
                        precision=lax.Precision.HIGHEST)
    hsT_ref[...] = h * dinv


def _mid_body(p1_ref, hsT_ref, dinv_ref, w2t_ref, b1_ref, hs2T_ref):
    dinv = dinv_ref[...]
    agg = jnp.sum(p1_ref[...], axis=0)
    z = jnp.maximum(dinv * (agg + hsT_ref[...]) + b1_ref[...], 0.0)
    h2 = lax.dot_general(w2t_ref[...], z, (((1,), (0,)), ((), ())),
                         preferred_element_type=jnp.float32,
                        precision=lax.Precision.HIGHEST)
    hs2T_ref[...] = h2 * dinv


def _fin_body(p2_ref, hs2T_ref, dinv_ref, b2_ref, out_ref):
    agg = jnp.sum(p2_ref[...], axis=0)
    a = dinv_ref[...] * (agg + hs2T_ref[...]) + b2_ref[...]
    m = jnp.max(a, axis=0, keepdims=True)
    lse = m + jnp.log(jnp.sum(jnp.exp(a - m), axis=0, keepdims=True))
    out_ref[...] = a - lse


def kernel(x, edge_index, edge_weight, W1, b1, W2, b2):
    src = edge_index[0].astype(jnp.int32)
    dst = edge_index[1].astype(jnp.int32)
    ew = edge_weight.astype(jnp.float32)

    deg_part = _deg_kernel(dst, ew)

    dinv, hsT = pl.pallas_call(
        _prep_body,
        out_shape=(
            jax.ShapeDtypeStruct((1, N), jnp.float32),
            jax.ShapeDtypeStruct((H1, N), jnp.float32),
        ),
    )(deg_part, x, W1.T)

    p1 = _agg1_kernel(hsT.reshape(H1 * N), src, dst, ew)

    hs2T = pl.pallas_call(
        _mid_body,
        out_shape=jax.ShapeDtypeStruct((H2, N), jnp.float32),
    )(p1.reshape(_NC1, H1, N), hsT, dinv, W2.T, b1.reshape(H1, 1))

    p2 = _agg2_kernel(hs2T.reshape(H2 * N), src, dst, ew)

    outT = pl.pallas_call(
        _fin_body,
        out_shape=jax.ShapeDtypeStruct((H2, N), jnp.float32),
    )(p2.reshape(NW, H2, N), hs2T, dinv, b2.reshape(H2, 1))

    return outT.T


# speedup vs baseline: 1.0529x; 1.0529x over previous
import functools

import jax
import jax.numpy as jnp
from jax import lax
from jax.experimental import pallas as pl
from jax.experimental.pallas import tpu as pltpu
from jax.experimental.pallas import tpu_sc as plsc

N = 10000
E = 320000
D = 128
H1 = 16
H2 = 2
NW = 32
LANES = 16

_MESH = plsc.VectorSubcoreMesh(core_axis_name="c", subcore_axis_name="s")


def _wid():
    return lax.axis_index("s") * 2 + lax.axis_index("c")


def _zero_vmem(ref, n):
    @plsc.parallel_loop(0, n // LANES, unroll=8)
    def _(i):
        ref[pl.ds(i * LANES, LANES)] = jnp.zeros((LANES,), jnp.float32)


_EPT = E // NW


@functools.partial(
    pl.kernel,
    out_type=jax.ShapeDtypeStruct((NW, N), jnp.float32),
    mesh=_MESH,
    compiler_params=pltpu.CompilerParams(needs_layout_passes=False),
    scratch_types=[
        pltpu.VMEM((_EPT,), jnp.int32),
        pltpu.VMEM((_EPT,), jnp.float32),
        pltpu.VMEM((N,), jnp.float32),
    ],
)
def _deg_kernel(dst_hbm, ew_hbm, out_hbm, dst_v, ew_v, acc_v):
    w = _wid()
    pltpu.sync_copy(dst_hbm.at[pl.ds(w * _EPT, _EPT)], dst_v)
    pltpu.sync_copy(ew_hbm.at[pl.ds(w * _EPT, _EPT)], ew_v)
    _zero_vmem(acc_v, N)

    @plsc.parallel_loop(0, _EPT // LANES, unroll=8)
    def _(i):
        d = dst_v[pl.ds(i * LANES, LANES)]
        v = ew_v[pl.ds(i * LANES, LANES)]
        plsc.addupdate_scatter(acc_v, [d], v)

    pltpu.sync_copy(acc_v, out_hbm.at[w])


_FG1 = 4
_NC1 = H1 // _FG1 * 2
_EC1 = E // _NC1
_CH1 = 4000
_NSUB1 = _EC1 // _CH1
assert _EC1 % _CH1 == 0 and _CH1 % LANES == 0


def _edge_loop_fg(nvec, fg, src_v, dst_v, ew_v, h_v, acc_v, unroll=4):
    @plsc.parallel_loop(0, nvec, unroll=unroll)
    def _(i):
        s = src_v[pl.ds(i * LANES, LANES)]
        d = dst_v[pl.ds(i * LANES, LANES)]
        v = ew_v[pl.ds(i * LANES, LANES)]
        for f in range(fg):
            g = plsc.load_gather(h_v, [s + f * N])
            plsc.addupdate_scatter(acc_v, [d + f * N], g * v)


@functools.partial(
    pl.kernel,
    out_type=jax.ShapeDtypeStruct((_NC1 * H1 * N,), jnp.float32),
    mesh=_MESH,
    compiler_params=pltpu.CompilerParams(needs_layout_passes=False),
    scratch_types=[
        pltpu.VMEM((_FG1 * N,), jnp.float32),
        pltpu.VMEM((_FG1 * N,), jnp.float32),
        pltpu.VMEM((2 * _CH1,), jnp.int32),
        pltpu.VMEM((2 * _CH1,), jnp.int32),
        pltpu.VMEM((2 * _CH1,), jnp.float32),
        pltpu.SemaphoreType.DMA,
        pltpu.SemaphoreType.DMA,
    ],
)
def _agg1_kernel(hsT_hbm, src_hbm, dst_hbm, ew_hbm, out_hbm,
                 h_v, acc_v, src_v, dst_v, ew_v, sem0, sem1):
    w = _wid()
    g = w % _FG1
    chunk = w // _FG1
    base = chunk * _EC1
    sems = (sem0, sem1)

    def fire(c):
        b = c % 2
        off = base + c * _CH1
        sl = pl.ds(b * _CH1, _CH1)
        return [
            pltpu.async_copy(src_hbm.at[pl.ds(off, _CH1)], src_v.at[sl], sems[b]),
            pltpu.async_copy(dst_hbm.at[pl.ds(off, _CH1)], dst_v.at[sl], sems[b]),
            pltpu.async_copy(ew_hbm.at[pl.ds(off, _CH1)], ew_v.at[sl], sems[b]),
        ]

    pending = {0: fire(0)}
    pltpu.sync_copy(hsT_hbm.at[pl.ds(g * _FG1 * N, _FG1 * N)], h_v)
    _zero_vmem(acc_v, _FG1 * N)
    for c in range(_NSUB1):
        if c + 1 < _NSUB1:
            pending[(c + 1) % 2] = fire(c + 1)
        for desc in pending[c % 2]:
            desc.wait()
        sl = pl.ds((c % 2) * _CH1, _CH1)
        _edge_loop_fg(_CH1 // LANES, _FG1,
                      src_v.at[sl], dst_v.at[sl], ew_v.at[sl], h_v, acc_v)
    pltpu.sync_copy(
        acc_v, out_hbm.at[pl.ds(chunk * H1 * N + g * _FG1 * N, _FG1 * N)])


_ECH = E // NW


@functools.partial(
    pl.kernel,
    out_type=jax.ShapeDtypeStruct((NW * H2 * N,), jnp.float32),
    mesh=_MESH,
    compiler_params=pltpu.CompilerParams(needs_layout_passes=False),
    scratch_types=[
        pltpu.VMEM((H2 * N,), jnp.float32),
        pltpu.VMEM((H2 * N,), jnp.float32),
        pltpu.VMEM((_ECH,), jnp.int32),
        pltpu.VMEM((_ECH,), jnp.int32),
        pltpu.VMEM((_ECH,), jnp.float32),
        pltpu.SemaphoreType.DMA,
    ],
)
def _agg2_kernel(hs2T_hbm, src_hbm, dst_hbm, ew_hbm, out_hbm,
                 h_v, acc_v, src_v, dst_v, ew_v, sem):
    w = _wid()
    off = w * _ECH
    pending = [
        pltpu.async_copy(src_hbm.at[pl.ds(off, _ECH)], src_v, sem),
        pltpu.async_copy(dst_hbm.at[pl.ds(off, _ECH)], dst_v, sem),
        pltpu.async_copy(ew_hbm.at[pl.ds(off, _ECH)], ew_v, sem),
    ]
    pltpu.sync_copy(hs2T_hbm, h_v)
    _zero_vmem(acc_v, H2 * N)
    for desc in pending:
        desc.wait()
    _edge_loop_fg(_ECH // LANES, H2, src_v, dst_v, ew_v, h_v, acc_v, unroll=8)
    pltpu.sync_copy(acc_v, out_hbm.at[pl.ds(w * H2 * N, H2 * N)])


def _mm_body(x_ref, w1t_ref, hT_ref):
    hT_ref[...] = lax.dot_general(
        w1t_ref[...], x_ref[...], (((1,), (1,)), ((), ())),
        preferred_element_type=jnp.float32, precision=lax.Precision.HIGHEST)


def _scale_body(degp_ref, hT_ref, dinv_ref, hsT_ref):
    deg = jnp.sum(degp_ref[...], axis=0, keepdims=True) + 1.0
    dinv = jnp.where(deg > 0, lax.rsqrt(jnp.maximum(deg, 1e-12)), 0.0)
    dinv_ref[...] = dinv
    hsT_ref[...] = hT_ref[...] * dinv


def _mid_body(p1_ref, hsT_ref, dinv_ref, w2t_ref, b1_ref, hs2T_ref):
    dinv = dinv_ref[...]
    agg = jnp.sum(p1_ref[...], axis=0)
    z = jnp.maximum(dinv * (agg + hsT_ref[...]) + b1_ref[...], 0.0)
    h2 = lax.dot_general(w2t_ref[...], z, (((1,), (0,)), ((), ())),
                         preferred_element_type=jnp.float32,
                        precision=lax.Precision.HIGHEST)
    hs2T_ref[...] = h2 * dinv


def _fin_body(p2_ref, hs2T_ref, dinv_ref, b2_ref, out_ref):
    agg = jnp.sum(p2_ref[...], axis=0)
    a = dinv_ref[...] * (agg + hs2T_ref[...]) + b2_ref[...]
    m = jnp.max(a, axis=0, keepdims=True)
    lse = m + jnp.log(jnp.sum(jnp.exp(a - m), axis=0, keepdims=True))
    out_ref[...] = a - lse


def kernel(x, edge_index, edge_weight, W1, b1, W2, b2):
    src = edge_index[0].astype(jnp.int32)
    dst = edge_index[1].astype(jnp.int32)
    ew = edge_weight.astype(jnp.float32)

    hT = pl.pallas_call(
        _mm_body,
        out_shape=jax.ShapeDtypeStruct((H1, N), jnp.float32),
    )(x, W1.T)
    deg_part = _deg_kernel(dst, ew)

    dinv, hsT = pl.pallas_call(
        _scale_body,
        out_shape=(
            jax.ShapeDtypeStruct((1, N), jnp.float32),
            jax.ShapeDtypeStruct((H1, N), jnp.float32),
        ),
    )(deg_part, hT)

    p1 = _agg1_kernel(hsT.reshape(H1 * N), src, dst, ew)

    hs2T = pl.pallas_call(
        _mid_body,
        out_shape=jax.ShapeDtypeStruct((H2, N), jnp.float32),
    )(p1.reshape(_NC1, H1, N), hsT, dinv, W2.T, b1.reshape(H1, 1))

    p2 = _agg2_kernel(hs2T.reshape(H2 * N), src, dst, ew)

    outT = pl.pallas_call(
        _fin_body,
        out_shape=jax.ShapeDtypeStruct((H2, N), jnp.float32),
    )(p2.reshape(NW, H2, N), hs2T, dinv, b2.reshape(H2, 1))

    return outT.T


# speedup vs baseline: 1.0655x; 1.0120x over previous
import functools

import jax
import jax.numpy as jnp
from jax import lax
from jax.experimental import pallas as pl
from jax.experimental.pallas import tpu as pltpu
from jax.experimental.pallas import tpu_sc as plsc

N = 10000
E = 320000
D = 128
H1 = 16
H2 = 2
NW = 32
LANES = 16

_MESH = plsc.VectorSubcoreMesh(core_axis_name="c", subcore_axis_name="s")


def _wid():
    return lax.axis_index("s") * 2 + lax.axis_index("c")


def _zero_vmem(ref, n):
    @plsc.parallel_loop(0, n // LANES, unroll=8)
    def _(i):
        ref[pl.ds(i * LANES, LANES)] = jnp.zeros((LANES,), jnp.float32)


def _zero_vmem2(ref, fg, n):
    @plsc.parallel_loop(0, n // LANES, unroll=8)
    def _(i):
        for f in range(fg):
            ref[f, pl.ds(i * LANES, LANES)] = jnp.zeros((LANES,), jnp.float32)


_EPT = E // NW


@functools.partial(
    pl.kernel,
    out_type=jax.ShapeDtypeStruct((NW, N), jnp.float32),
    mesh=_MESH,
    compiler_params=pltpu.CompilerParams(needs_layout_passes=False),
    scratch_types=[
        pltpu.VMEM((_EPT,), jnp.int32),
        pltpu.VMEM((_EPT,), jnp.float32),
        pltpu.VMEM((N,), jnp.float32),
    ],
)
def _deg_kernel(dst_hbm, ew_hbm, out_hbm, dst_v, ew_v, acc_v):
    w = _wid()
    pltpu.sync_copy(dst_hbm.at[pl.ds(w * _EPT, _EPT)], dst_v)
    pltpu.sync_copy(ew_hbm.at[pl.ds(w * _EPT, _EPT)], ew_v)
    _zero_vmem(acc_v, N)

    @plsc.parallel_loop(0, _EPT // LANES, unroll=8)
    def _(i):
        d = dst_v[pl.ds(i * LANES, LANES)]
        v = ew_v[pl.ds(i * LANES, LANES)]
        plsc.addupdate_scatter(acc_v, [d], v)

    pltpu.sync_copy(acc_v, out_hbm.at[w])


_FG1 = 4
_NC1 = H1 // _FG1 * 2
_EC1 = E // _NC1
_CH1 = 4000
_NSUB1 = _EC1 // _CH1
assert _EC1 % _CH1 == 0 and _CH1 % LANES == 0


def _edge_loop_fg(nvec, fg, src_v, dst_v, ew_v, h_v, acc_v, unroll=4):
    fvecs = [jnp.full((LANES,), f, jnp.int32) for f in range(fg)]

    @plsc.parallel_loop(0, nvec, unroll=unroll)
    def _(i):
        s = src_v[pl.ds(i * LANES, LANES)]
        d = dst_v[pl.ds(i * LANES, LANES)]
        v = ew_v[pl.ds(i * LANES, LANES)]
        for f in range(fg):
            g = plsc.load_gather(h_v, [fvecs[f], s])
            plsc.addupdate_scatter(acc_v, [fvecs[f], d], g * v)


@functools.partial(
    pl.kernel,
    out_type=jax.ShapeDtypeStruct((_NC1, H1 // _FG1, _FG1, N), jnp.float32),
    mesh=_MESH,
    compiler_params=pltpu.CompilerParams(needs_layout_passes=False),
    scratch_types=[
        pltpu.VMEM((_FG1, N), jnp.float32),
        pltpu.VMEM((_FG1, N), jnp.float32),
        pltpu.VMEM((2 * _CH1,), jnp.int32),
        pltpu.VMEM((2 * _CH1,), jnp.int32),
        pltpu.VMEM((2 * _CH1,), jnp.float32),
        pltpu.SemaphoreType.DMA,
        pltpu.SemaphoreType.DMA,
    ],
)
def _agg1_kernel(hsT_hbm, src_hbm, dst_hbm, ew_hbm, out_hbm,
                 h_v, acc_v, src_v, dst_v, ew_v, sem0, sem1):
    w = _wid()
    g = w % _FG1
    chunk = w // _FG1
    base = chunk * _EC1
    sems = (sem0, sem1)

    def fire(c):
        b = c % 2
        off = base + c * _CH1
        sl = pl.ds(b * _CH1, _CH1)
        return [
            pltpu.async_copy(src_hbm.at[pl.ds(off, _CH1)], src_v.at[sl], sems[b]),
            pltpu.async_copy(dst_hbm.at[pl.ds(off, _CH1)], dst_v.at[sl], sems[b]),
            pltpu.async_copy(ew_hbm.at[pl.ds(off, _CH1)], ew_v.at[sl], sems[b]),
        ]

    pending = {0: fire(0)}
    pltpu.sync_copy(hsT_hbm.at[g], h_v)
    _zero_vmem2(acc_v, _FG1, N)
    for c in range(_NSUB1):
        if c + 1 < _NSUB1:
            pending[(c + 1) % 2] = fire(c + 1)
        for desc in pending[c % 2]:
            desc.wait()
        sl = pl.ds((c % 2) * _CH1, _CH1)
        _edge_loop_fg(_CH1 // LANES, _FG1,
                      src_v.at[sl], dst_v.at[sl], ew_v.at[sl], h_v, acc_v)
    pltpu.sync_copy(acc_v, out_hbm.at[chunk, g])


_ECH = E // NW


@functools.partial(
    pl.kernel,
    out_type=jax.ShapeDtypeStruct((NW, H2, N), jnp.float32),
    mesh=_MESH,
    compiler_params=pltpu.CompilerParams(needs_layout_passes=False),
    scratch_types=[
        pltpu.VMEM((H2, N), jnp.float32),
        pltpu.VMEM((H2, N), jnp.float32),
        pltpu.VMEM((_ECH,), jnp.int32),
        pltpu.VMEM((_ECH,), jnp.int32),
        pltpu.VMEM((_ECH,), jnp.float32),
        pltpu.SemaphoreType.DMA,
    ],
)
def _agg2_kernel(hs2T_hbm, src_hbm, dst_hbm, ew_hbm, out_hbm,
                 h_v, acc_v, src_v, dst_v, ew_v, sem):
    w = _wid()
    off = w * _ECH
    pending = [
        pltpu.async_copy(src_hbm.at[pl.ds(off, _ECH)], src_v, sem),
        pltpu.async_copy(dst_hbm.at[pl.ds(off, _ECH)], dst_v, sem),
        pltpu.async_copy(ew_hbm.at[pl.ds(off, _ECH)], ew_v, sem),
    ]
    pltpu.sync_copy(hs2T_hbm.at[0], h_v)
    _zero_vmem2(acc_v, H2, N)
    for desc in pending:
        desc.wait()
    _edge_loop_fg(_ECH // LANES, H2, src_v, dst_v, ew_v, h_v, acc_v, unroll=8)
    pltpu.sync_copy(acc_v, out_hbm.at[w])


_NG = H1 // _FG1


def _mm_body(x_ref, w1t_ref, hT_ref):
    for g in range(_NG):
        hT_ref[g] = lax.dot_general(
            w1t_ref[g], x_ref[...], (((1,), (1,)), ((), ())),
            preferred_element_type=jnp.float32,
            precision=lax.Precision.HIGHEST)


def _scale_body(degp_ref, hT_ref, dinv_ref, hsT_ref):
    deg = jnp.sum(degp_ref[...], axis=0, keepdims=True) + 1.0
    dinv = jnp.where(deg > 0, lax.rsqrt(jnp.maximum(deg, 1e-12)), 0.0)
    dinv_ref[...] = dinv
    hsT_ref[...] = hT_ref[...] * dinv[None]


def _mid_body(p1_ref, hsT_ref, dinv_ref, w2t_ref, b1_ref, hs2T_ref):
    dinv = dinv_ref[...]
    agg = jnp.sum(p1_ref[...], axis=0)
    z = jnp.maximum(dinv[None] * (agg + hsT_ref[...]) + b1_ref[...], 0.0)
    h2 = jnp.zeros((H2, N), jnp.float32)
    for g in range(_NG):
        h2 = h2 + lax.dot_general(
            w2t_ref[g], z[g], (((1,), (0,)), ((), ())),
            preferred_element_type=jnp.float32,
            precision=lax.Precision.HIGHEST)
    hs2T_ref[0] = h2 * dinv


def _fin_body(p2_ref, hs2T_ref, dinv_ref, b2_ref, out_ref):
    agg = jnp.sum(p2_ref[...], axis=0)
    a = dinv_ref[...] * (agg + hs2T_ref[0]) + b2_ref[...]
    m = jnp.max(a, axis=0, keepdims=True)
    lse = m + jnp.log(jnp.sum(jnp.exp(a - m), axis=0, keepdims=True))
    out_ref[...] = a - lse


def kernel(x, edge_index, edge_weight, W1, b1, W2, b2):
    src = edge_index[0].astype(jnp.int32)
    dst = edge_index[1].astype(jnp.int32)
    ew = edge_weight.astype(jnp.float32)

    hT = pl.pallas_call(
        _mm_body,
        out_shape=jax.ShapeDtypeStruct((_NG, _FG1, N), jnp.float32),
    )(x, W1.T.reshape(_NG, _FG1, D))
    deg_part = _deg_kernel(dst, ew)

    dinv, hsT = pl.pallas_call(
        _scale_body,
        out_shape=(
            jax.ShapeDtypeStruct((1, N), jnp.float32),
            jax.ShapeDtypeStruct((_NG, _FG1, N), jnp.float32),
        ),
    )(deg_part, hT)

    p1 = _agg1_kernel(hsT, src, dst, ew)

    hs2T = pl.pallas_call(
        _mid_body,
        out_shape=jax.ShapeDtypeStruct((1, H2, N), jnp.float32),
    )(p1, hsT, dinv, W2.T.reshape(H2, _NG, _FG1).transpose(1, 0, 2),
      b1.reshape(_NG, _FG1, 1))

    p2 = _agg2_kernel(hs2T, src, dst, ew)

    outT = pl.pallas_call(
        _fin_body,
        out_shape=jax.ShapeDtypeStruct((H2, N), jnp.float32),
    )(p2, hs2T, dinv, b2.reshape(H2, 1))

    return outT.T
